# baseline (speedup 1.0000x reference)
import functools

import jax
import jax.numpy as jnp
from jax import lax
from jax.experimental import pallas as pl
from jax.experimental.pallas import tpu as pltpu
from jax.experimental.pallas import tpu_sc as plsc

NC = 2
NS = 16
CHUNK = 128
NCHUNK = 79
CH2 = 64
NCH2 = 158
DEGW = 128
ZROWS = 640


def _cdiv(a, b):
    return (a + b - 1) // b


@functools.lru_cache(maxsize=None)
def _sc_calls(N, E, D):
    n_tiles = NC * NS
    nch = NCHUNK
    e_pad = nch * CHUNK * n_tiles
    acc_rows = ZROWS * NS
    assert e_pad >= E and N % NS == 0 and acc_rows >= N + 1

    mesh = plsc.VectorSubcoreMesh(core_axis_name="c", subcore_axis_name="s")

    @functools.partial(
        pl.kernel,
        out_type=jax.ShapeDtypeStruct((NC, ZROWS * NS, DEGW), jnp.float32),
        mesh=mesh,
        scratch_types=[
            pltpu.VMEM((nch, CHUNK), jnp.int32),
            pltpu.VMEM((CHUNK, DEGW), jnp.float32),
            pltpu.VMEM_SHARED((acc_rows, DEGW), jnp.float32),
        ],
    )
    def deg_call(dst_hbm, zeros_hbm, ones_hbm, out_hbm, dst_v, ones_v, acc_sh):
        c = lax.axis_index("c")
        s = lax.axis_index("s")
        pltpu.sync_copy(zeros_hbm, acc_sh.at[pl.ds(s * ZROWS, ZROWS)])
        pltpu.sync_copy(ones_hbm, ones_v)
        pltpu.sync_copy(dst_hbm.at[c, s], dst_v)
        plsc.subcore_barrier()

        def body(j, carry):
            pltpu.sync_copy(ones_v, acc_sh.at[dst_v.at[j]], add=True)
            return carry

        lax.fori_loop(0, nch, body, 0)
        plsc.subcore_barrier()
        pltpu.sync_copy(acc_sh.at[pl.ds(s * ZROWS, ZROWS)],
                        out_hbm.at[c, pl.ds(s * ZROWS, ZROWS)])

    @functools.partial(
        pl.kernel,
        out_type=jax.ShapeDtypeStruct((NC, ZROWS * NS, D), jnp.float32),
        mesh=mesh,
        scratch_types=[
            pltpu.VMEM((NCH2 * CH2,), jnp.int32),
            pltpu.VMEM((NCH2 * CH2,), jnp.int32),
            pltpu.VMEM((CH2, D), jnp.float32),
            pltpu.VMEM((CH2, D), jnp.float32),
            pltpu.VMEM((CH2, D), jnp.float32),
            pltpu.VMEM_SHARED((acc_rows, D), jnp.float32),
            pltpu.SemaphoreType.DMA,
            pltpu.SemaphoreType.DMA,
            pltpu.SemaphoreType.DMA,
        ],
    )
    def scat_call(g_hbm, src_hbm, dst_hbm, zeros_hbm, out_hbm,
                  src_v, dst_v, rows_a, rows_b, rows_c, acc_sh,
                  sem_a, sem_b, sem_c):
        c = lax.axis_index("c")
        s = lax.axis_index("s")
        pltpu.sync_copy(zeros_hbm, acc_sh.at[pl.ds(s * ZROWS, ZROWS)])
        pltpu.sync_copy(src_hbm.at[c, s], src_v)
        pltpu.sync_copy(dst_hbm.at[c, s], dst_v)
        plsc.subcore_barrier()

        def gather(j, buf, sem):
            return pltpu.async_copy(
                g_hbm.at[src_v.at[pl.ds(j * CH2, CH2)]], buf, sem)

        def gwait(j, buf, sem):
            pltpu.make_async_copy(
                g_hbm.at[src_v.at[pl.ds(j * CH2, CH2)]], buf, sem).wait()

        def scatter(j, buf):
            pltpu.sync_copy(buf, acc_sh.at[dst_v.at[pl.ds(j * CH2, CH2)]],
                            add=True)

        bufs = ((rows_a, sem_a), (rows_b, sem_b), (rows_c, sem_c))
        gather(0, rows_a, sem_a)
        gather(1, rows_b, sem_b)

        def body(t, carry):
            j = 3 * t
            for k in range(3):
                buf, sem = bufs[k]
                nbuf, nsem = bufs[(k + 2) % 3]
                gwait(j + k, buf, sem)

                @pl.when(j + k + 2 < NCH2)
                def _():
                    gather(j + k + 2, nbuf, nsem)

                scatter(j + k, buf)
            return carry

        lax.fori_loop(0, NCH2 // 3, body, 0)
        for j in range(NCH2 - NCH2 % 3, NCH2):
            buf, sem = bufs[j % 3]
            gwait(j, buf, sem)
            scatter(j, buf)
        plsc.subcore_barrier()
        pltpu.sync_copy(acc_sh.at[pl.ds(s * ZROWS, ZROWS)],
                        out_hbm.at[c, pl.ds(s * ZROWS, ZROWS)])

    return deg_call, scat_call, e_pad


@functools.lru_cache(maxsize=None)
def _tc_calls(N, D):
    BR = 2000 if N % 2000 == 0 else N // NS
    grid = (N // BR,)
    xb = pl.BlockSpec((BR, D), lambda b: (b, 0))
    wb = pl.BlockSpec((D, D), lambda b: (0, 0))
    bb = pl.BlockSpec((1, D), lambda b: (0, 0))
    db = pl.BlockSpec((NC, BR, DEGW), lambda b: (0, b, 0))
    sb = pl.BlockSpec((NC, BR, D), lambda b: (0, b, 0))
    oshape = jax.ShapeDtypeStruct((N, D), jnp.float32)

    def dinv_of(d_ref):
        return lax.rsqrt(d_ref[0, :, 0:1] + d_ref[1, :, 0:1] + 1.0)

    def k1_body(x_ref, w_ref, d_ref, o_ref):
        h = jnp.dot(x_ref[...], w_ref[...], preferred_element_type=jnp.float32)
        o_ref[...] = h * dinv_of(d_ref)

    k1 = pl.pallas_call(k1_body, grid=grid, in_specs=[xb, wb, db],
                        out_specs=xb, out_shape=oshape)

    def k3_body(s_ref, g_ref, d_ref, b_ref, w_ref, o_ref):
        dinv = dinv_of(d_ref)
        x = (s_ref[0] + s_ref[1] + g_ref[...]) * dinv + b_ref[...]
        x = jnp.maximum(x, 0.0)
        o_ref[...] = jnp.dot(x, w_ref[...],
                             preferred_element_type=jnp.float32) * dinv

    k3 = pl.pallas_call(k3_body, grid=grid, in_specs=[sb, xb, db, bb, wb],
                        out_specs=xb, out_shape=oshape)

    def k5_body(s_ref, g_ref, d_ref, b_ref, o_ref):
        o_ref[...] = ((s_ref[0] + s_ref[1] + g_ref[...]) * dinv_of(d_ref)
                      + b_ref[...])

    k5 = pl.pallas_call(k5_body, grid=grid, in_specs=[sb, xb, db, bb],
                        out_specs=xb, out_shape=oshape)

    return k1, k3, k5


def kernel(features, edge_index, W1, b1, W2, b2):
    N, D = features.shape
    E = edge_index.shape[1]
    deg_call, scat_call, e_pad = _sc_calls(N, E, D)
    k1, k3, k5 = _tc_calls(N, D)

    src = edge_index[0].astype(jnp.int32)
    dst = edge_index[1].astype(jnp.int32)
    pad = e_pad - E
    dump = N + jnp.arange(pad, dtype=jnp.int32) % (ZROWS * NS - N)
    src_p = jnp.concatenate([src, jnp.zeros((pad,), jnp.int32)])
    dst_p = jnp.concatenate([dst, dump])
    dst_deg = dst_p.reshape(NC, NS, NCHUNK, CHUNK)
    src_r = src_p.reshape(NC, NS, NCH2 * CH2)
    dst_r = dst_p.reshape(NC, NS, NCH2 * CH2)
    zeros_h = jnp.zeros((ZROWS, D), jnp.float32)

    ones_h = jnp.ones((CHUNK, DEGW), jnp.float32)
    b1r = b1.reshape(1, D)
    b2r = b2.reshape(1, D)

    deg = deg_call(dst_deg, zeros_h, ones_h)
    g1 = k1(features, W1, deg)
    s1 = scat_call(g1, src_r, dst_r, zeros_h)
    g2 = k3(s1, g1, deg, b1r, W2)
    s2 = scat_call(g2, src_r, dst_r, zeros_h)
    return k5(s2, g2, deg, b2r)


# speedup vs baseline: 1.0561x; 1.0561x over previous
import functools

import jax
import jax.numpy as jnp
from jax import lax
from jax.experimental import pallas as pl
from jax.experimental.pallas import tpu as pltpu
from jax.experimental.pallas import tpu_sc as plsc

NC = 2
NS = 16
CHUNK = 128
NCHUNK = 79
NPH = 2
PH = 40
DEGW = 128
ZROWS = 640


def _cdiv(a, b):
    return (a + b - 1) // b


@functools.lru_cache(maxsize=None)
def _sc_calls(N, E, D):
    n_tiles = NC * NS
    nch = NCHUNK
    e_pad = nch * CHUNK * n_tiles
    acc_rows = ZROWS * NS
    assert e_pad >= E and N % NS == 0 and acc_rows >= N + 1

    mesh = plsc.VectorSubcoreMesh(core_axis_name="c", subcore_axis_name="s")

    @functools.partial(
        pl.kernel,
        out_type=jax.ShapeDtypeStruct((NC, ZROWS * NS, DEGW), jnp.float32),
        mesh=mesh,
        scratch_types=[
            pltpu.VMEM((nch, CHUNK), jnp.int32),
            pltpu.VMEM((CHUNK, DEGW), jnp.float32),
            pltpu.VMEM_SHARED((acc_rows, DEGW), jnp.float32),
        ],
    )
    def deg_call(dst_hbm, zeros_hbm, ones_hbm, out_hbm, dst_v, ones_v, acc_sh):
        c = lax.axis_index("c")
        s = lax.axis_index("s")
        pltpu.sync_copy(zeros_hbm, acc_sh.at[pl.ds(s * ZROWS, ZROWS)])
        pltpu.sync_copy(ones_hbm, ones_v)
        pltpu.sync_copy(dst_hbm.at[c, s], dst_v)
        plsc.subcore_barrier()

        def body(j, carry):
            pltpu.sync_copy(ones_v, acc_sh.at[dst_v.at[j]], add=True)
            return carry

        lax.fori_loop(0, nch, body, 0)
        plsc.subcore_barrier()
        pltpu.sync_copy(acc_sh.at[pl.ds(s * ZROWS, ZROWS)],
                        out_hbm.at[c, pl.ds(s * ZROWS, ZROWS)])

    @functools.partial(
        pl.kernel,
        out_type=jax.ShapeDtypeStruct((NC, ZROWS * NS, D), jnp.float32),
        mesh=mesh,
        scratch_types=[
            pltpu.VMEM((PH, CHUNK), jnp.int32),
            pltpu.VMEM((PH, CHUNK), jnp.int32),
            pltpu.VMEM((CHUNK, D), jnp.float32),
            pltpu.VMEM((CHUNK, D), jnp.float32),
            pltpu.VMEM_SHARED((acc_rows, D), jnp.float32),
            pltpu.SemaphoreType.DMA,
            pltpu.SemaphoreType.DMA,
        ],
    )
    def scat_call(g_hbm, src_hbm, dst_hbm, zeros_hbm, out_hbm,
                  src_v, dst_v, rows_a, rows_b, acc_sh, sem_a, sem_b):
        c = lax.axis_index("c")
        s = lax.axis_index("s")
        pltpu.sync_copy(zeros_hbm, acc_sh.at[pl.ds(s * ZROWS, ZROWS)])
        plsc.subcore_barrier()

        def gather(j, buf, sem):
            return pltpu.async_copy(g_hbm.at[src_v.at[j]], buf, sem)

        def gwait(j, buf, sem):
            pltpu.make_async_copy(g_hbm.at[src_v.at[j]], buf, sem).wait()

        def scatter(j, buf):
            pltpu.sync_copy(buf, acc_sh.at[dst_v.at[j]], add=True)

        for ph in range(NPH):
            nc_ph = min(PH, nch - ph * PH)
            pltpu.sync_copy(src_hbm.at[c, s, pl.ds(ph * PH, PH)], src_v)
            pltpu.sync_copy(dst_hbm.at[c, s, pl.ds(ph * PH, PH)], dst_v)
            gather(0, rows_a, sem_a)

            def body(t, carry):
                j0 = 2 * t
                j1 = 2 * t + 1
                gwait(j0, rows_a, sem_a)
                gather(j1, rows_b, sem_b)
                scatter(j0, rows_a)
                gwait(j1, rows_b, sem_b)

                @pl.when(j1 + 1 < nc_ph)
                def _():
                    gather(j1 + 1, rows_a, sem_a)

                scatter(j1, rows_b)
                return carry

            lax.fori_loop(0, nc_ph // 2, body, 0)
            if nc_ph % 2:
                jt = nc_ph - 1
                gwait(jt, rows_a, sem_a)
                scatter(jt, rows_a)
        plsc.subcore_barrier()
        pltpu.sync_copy(acc_sh.at[pl.ds(s * ZROWS, ZROWS)],
                        out_hbm.at[c, pl.ds(s * ZROWS, ZROWS)])

    return deg_call, scat_call, e_pad


@functools.lru_cache(maxsize=None)
def _tc_calls(N, D):
    BR = 2000 if N % 2000 == 0 else N // NS
    grid = (N // BR,)
    xb = pl.BlockSpec((BR, D), lambda b: (b, 0))
    wb = pl.BlockSpec((D, D), lambda b: (0, 0))
    bb = pl.BlockSpec((1, D), lambda b: (0, 0))
    db = pl.BlockSpec((NC, BR, DEGW), lambda b: (0, b, 0))
    sb = pl.BlockSpec((NC, BR, D), lambda b: (0, b, 0))
    oshape = jax.ShapeDtypeStruct((N, D), jnp.float32)

    def dinv_of(d_ref):
        return lax.rsqrt(d_ref[0, :, 0:1] + d_ref[1, :, 0:1] + 1.0)

    def k1_body(x_ref, w_ref, d_ref, o_ref):
        h = jnp.dot(x_ref[...], w_ref[...], preferred_element_type=jnp.float32)
        o_ref[...] = h * dinv_of(d_ref)

    k1 = pl.pallas_call(k1_body, grid=grid, in_specs=[xb, wb, db],
                        out_specs=xb, out_shape=oshape)

    def k3_body(s_ref, g_ref, d_ref, b_ref, w_ref, o_ref):
        dinv = dinv_of(d_ref)
        x = (s_ref[0] + s_ref[1] + g_ref[...]) * dinv + b_ref[...]
        x = jnp.maximum(x, 0.0)
        o_ref[...] = jnp.dot(x, w_ref[...],
                             preferred_element_type=jnp.float32) * dinv

    k3 = pl.pallas_call(k3_body, grid=grid, in_specs=[sb, xb, db, bb, wb],
                        out_specs=xb, out_shape=oshape)

    def k5_body(s_ref, g_ref, d_ref, b_ref, o_ref):
        o_ref[...] = ((s_ref[0] + s_ref[1] + g_ref[...]) * dinv_of(d_ref)
                      + b_ref[...])

    k5 = pl.pallas_call(k5_body, grid=grid, in_specs=[sb, xb, db, bb],
                        out_specs=xb, out_shape=oshape)

    return k1, k3, k5


def kernel(features, edge_index, W1, b1, W2, b2):
    N, D = features.shape
    E = edge_index.shape[1]
    deg_call, scat_call, e_pad = _sc_calls(N, E, D)
    k1, k3, k5 = _tc_calls(N, D)

    src = edge_index[0].astype(jnp.int32)
    dst = edge_index[1].astype(jnp.int32)
    pad = e_pad - E
    n_tiles = NC * NS
    dump = N + jnp.arange(pad, dtype=jnp.int32) % (ZROWS * NS - N)
    src79 = jnp.concatenate(
        [src, jnp.zeros((pad,), jnp.int32)]).reshape(n_tiles, NCHUNK, CHUNK)
    dst79 = jnp.concatenate([dst, dump]).reshape(n_tiles, NCHUNK, CHUNK)
    dst_deg = dst79.reshape(NC, NS, NCHUNK, CHUNK)
    fill = NPH * PH - NCHUNK
    src_r = jnp.concatenate(
        [src79, jnp.zeros((n_tiles, fill, CHUNK), jnp.int32)],
        axis=1).reshape(NC, NS, NPH * PH, CHUNK)
    dst_r = jnp.concatenate(
        [dst79, jnp.full((n_tiles, fill, CHUNK), N, jnp.int32)],
        axis=1).reshape(NC, NS, NPH * PH, CHUNK)
    zeros_h = jnp.zeros((ZROWS, D), jnp.float32)

    ones_h = jnp.ones((CHUNK, DEGW), jnp.float32)
    b1r = b1.reshape(1, D)
    b2r = b2.reshape(1, D)

    deg = deg_call(dst_deg, zeros_h, ones_h)
    g1 = k1(features, W1, deg)
    s1 = scat_call(g1, src_r, dst_r, zeros_h)
    g2 = k3(s1, g1, deg, b1r, W2)
    s2 = scat_call(g2, src_r, dst_r, zeros_h)
    return k5(s2, g2, deg, b2r)
